# Initial kernel scaffold; baseline (speedup 1.0000x reference)
import jax
import jax.numpy as jnp
from jax.experimental import pallas as pl


def kernel(x, Lp, W, b):
    raise NotImplementedError("write your pallas kernel here")


# speedup vs baseline: 2.8303x; 2.8303x over previous
import jax
import jax.numpy as jnp
from jax import lax
from jax.experimental import pallas as pl
from jax.experimental.pallas import tpu as pltpu
from jax.experimental.pallas import tpu_sc as plsc

L_TOTAL = 16384
VOCAB = 300
HIDDEN = 100
VPAD = 512
HPAD = 128
LANES = 16

NC = 2
NS = 16
NW = NC * NS
CHUNK = L_TOTAL // NW


def _hist_body(x_hbm, out_hbm, idx_v, hist_v):
    wid = lax.axis_index("s") * NC + lax.axis_index("c")
    base = wid * CHUNK
    pltpu.sync_copy(x_hbm.at[pl.ds(base, CHUNK)], idx_v)
    zeros = jnp.zeros((LANES,), jnp.float32)
    for i in range(VPAD // LANES):
        hist_v[pl.ds(i * LANES, LANES)] = zeros
    ones = jnp.ones((LANES,), jnp.float32)
    for i in range(CHUNK // LANES):
        idx = idx_v[pl.ds(i * LANES, LANES)]
        plsc.addupdate_scatter(hist_v, [idx], ones)
    pltpu.sync_copy(hist_v, out_hbm.at[wid])


_hist = pl.kernel(
    _hist_body,
    out_type=jax.ShapeDtypeStruct((NW, VPAD), jnp.float32),
    mesh=plsc.VectorSubcoreMesh(core_axis_name="c", subcore_axis_name="s"),
    scratch_types=[
        pltpu.VMEM((CHUNK,), jnp.int32),
        pltpu.VMEM((VPAD,), jnp.float32),
    ],
    compiler_params=pltpu.CompilerParams(needs_layout_passes=False),
)


def _dense_body(part_ref, lp_ref, w_ref, b_ref, out_ref):
    counts = jnp.sum(part_ref[...], axis=0, keepdims=True) * (1.0 / L_TOTAL)
    embed = jnp.dot(counts, lp_ref[...], preferred_element_type=jnp.float32)
    out = lax.dot_general(
        embed, w_ref[...], (((1,), (1,)), ((), ())),
        preferred_element_type=jnp.float32)
    out_ref[...] = out + b_ref[...]


_dense = pl.pallas_call(
    _dense_body,
    out_shape=jax.ShapeDtypeStruct((1, VPAD), jnp.float32),
)


def kernel(x, Lp, W, b):
    x = x.astype(jnp.int32)
    part = _hist(x)
    lp_pad = jnp.pad(Lp, ((0, VPAD - VOCAB), (0, HPAD - HIDDEN)))
    w_pad = jnp.pad(W, ((0, VPAD - VOCAB), (0, HPAD - HIDDEN)))
    b_pad = jnp.pad(b, (0, VPAD - VOCAB)).reshape(1, VPAD)
    out = _dense(part, lp_pad, w_pad, b_pad)
    return out[:, :VOCAB]


# speedup vs baseline: 3.0321x; 1.0713x over previous
import jax
import jax.numpy as jnp
from jax import lax
from jax.experimental import pallas as pl
from jax.experimental.pallas import tpu as pltpu
from jax.experimental.pallas import tpu_sc as plsc

L_TOTAL = 16384
VOCAB = 300
HIDDEN = 100
VPAD = 512
HPAD = 128
LANES = 16

NC = 2
NS = 16
NW = NC * NS
CHUNK = L_TOTAL // NW


def _hist_body(x_hbm, out_hbm, idx_v, hist_v):
    wid = lax.axis_index("s") * NC + lax.axis_index("c")
    base = wid * CHUNK
    pltpu.sync_copy(x_hbm.at[pl.ds(base, CHUNK)], idx_v)
    zeros = jnp.zeros((LANES,), jnp.float32)
    for i in range(VPAD // LANES):
        hist_v[pl.ds(i * LANES, LANES)] = zeros
    ones = jnp.ones((LANES,), jnp.float32)
    for i in range(CHUNK // LANES):
        idx = idx_v[pl.ds(i * LANES, LANES)]
        plsc.addupdate_scatter(hist_v, [idx], ones)
    pltpu.sync_copy(hist_v, out_hbm.at[wid])


_hist = pl.kernel(
    _hist_body,
    out_type=jax.ShapeDtypeStruct((NW, VPAD), jnp.float32),
    mesh=plsc.VectorSubcoreMesh(core_axis_name="c", subcore_axis_name="s"),
    scratch_types=[
        pltpu.VMEM((CHUNK,), jnp.int32),
        pltpu.VMEM((VPAD,), jnp.float32),
    ],
    compiler_params=pltpu.CompilerParams(needs_layout_passes=False),
)


def _dense_body(part_ref, lp_ref, w_ref, b_ref, out_ref):
    counts = jnp.sum(part_ref[:, :VOCAB], axis=0, keepdims=True) * (1.0 / L_TOTAL)
    embed = jnp.dot(counts, lp_ref[...], preferred_element_type=jnp.float32)
    out = lax.dot_general(
        embed, w_ref[...], (((1,), (1,)), ((), ())),
        preferred_element_type=jnp.float32)
    out_ref[...] = out + b_ref[...]


_dense = pl.pallas_call(
    _dense_body,
    out_shape=jax.ShapeDtypeStruct((1, VOCAB), jnp.float32),
)


def kernel(x, Lp, W, b):
    x = x.astype(jnp.int32)
    part = _hist(x)
    return _dense(part, Lp, W, b.reshape(1, VOCAB))


# speedup vs baseline: 3.0560x; 1.0079x over previous
import jax
import jax.numpy as jnp
from jax import lax
from jax.experimental import pallas as pl
from jax.experimental.pallas import tpu as pltpu
from jax.experimental.pallas import tpu_sc as plsc

L_TOTAL = 16384
VOCAB = 300
HIDDEN = 100
VPAD = 512
HPAD = 128
LANES = 16

NC = 2
NS = 16
NW = NC * NS
CHUNK = L_TOTAL // NW


def _hist_body(x_hbm, out_hbm, idx_v, hist_v):
    wid = lax.axis_index("s") * NC + lax.axis_index("c")
    base = wid * CHUNK
    pltpu.sync_copy(x_hbm.at[pl.ds(base, CHUNK)], idx_v)
    zeros = jnp.zeros((LANES,), jnp.float32)

    def zero_body(i, carry):
        hist_v[pl.ds(i * LANES, LANES)] = zeros
        return carry

    lax.fori_loop(0, VPAD // LANES, zero_body, 0)
    ones = jnp.ones((LANES,), jnp.float32)

    def scatter_body(i, carry):
        idx = idx_v[pl.ds(i * LANES, LANES)]
        plsc.addupdate_scatter(hist_v, [idx], ones)
        return carry

    lax.fori_loop(0, CHUNK // LANES, scatter_body, 0)
    pltpu.sync_copy(hist_v, out_hbm.at[wid])


_hist = pl.kernel(
    _hist_body,
    out_type=jax.ShapeDtypeStruct((NW, VPAD), jnp.float32),
    mesh=plsc.VectorSubcoreMesh(core_axis_name="c", subcore_axis_name="s"),
    scratch_types=[
        pltpu.VMEM((CHUNK,), jnp.int32),
        pltpu.VMEM((VPAD,), jnp.float32),
    ],
    compiler_params=pltpu.CompilerParams(needs_layout_passes=False),
)


def _dense_body(part_ref, lp_ref, w_ref, b_ref, out_ref):
    counts = jnp.sum(part_ref[:, :VOCAB], axis=0, keepdims=True) * (1.0 / L_TOTAL)
    embed = jnp.dot(counts, lp_ref[...], preferred_element_type=jnp.float32)
    out = lax.dot_general(
        embed, w_ref[...], (((1,), (1,)), ((), ())),
        preferred_element_type=jnp.float32)
    out_ref[...] = out + b_ref[...].reshape(1, VOCAB)


_dense = pl.pallas_call(
    _dense_body,
    out_shape=jax.ShapeDtypeStruct((1, VOCAB), jnp.float32),
)


def kernel(x, Lp, W, b):
    x = x.astype(jnp.int32)
    part = _hist(x)
    return _dense(part, Lp, W, b)


# speedup vs baseline: 3.2336x; 1.0581x over previous
import jax
import jax.numpy as jnp
from jax import lax
from jax.experimental import pallas as pl
from jax.experimental.pallas import tpu as pltpu
from jax.experimental.pallas import tpu_sc as plsc

L_TOTAL = 16384
VOCAB = 300
HIDDEN = 100
VPAD = 512
HPAD = 128
LANES = 16

NC = 1
NS = 16
NW = NC * NS
CHUNK = L_TOTAL // NW


def _hist_body(x_hbm, out_hbm, idx_v, hist_v):
    wid = lax.axis_index("s") * NC + lax.axis_index("c")
    base = wid * CHUNK
    pltpu.sync_copy(x_hbm.at[pl.ds(base, CHUNK)], idx_v)
    zeros = jnp.zeros((LANES,), jnp.float32)

    def zero_body(i, carry):
        hist_v[pl.ds(i * LANES, LANES)] = zeros
        return carry

    lax.fori_loop(0, VPAD // LANES, zero_body, 0)
    ones = jnp.ones((LANES,), jnp.float32)

    def scatter_body(i, carry):
        idx = idx_v[pl.ds(i * LANES, LANES)]
        plsc.addupdate_scatter(hist_v, [idx], ones)
        return carry

    lax.fori_loop(0, CHUNK // LANES, scatter_body, 0)
    pltpu.sync_copy(hist_v, out_hbm.at[wid])


_hist = pl.kernel(
    _hist_body,
    out_type=jax.ShapeDtypeStruct((NW, VPAD), jnp.float32),
    mesh=plsc.VectorSubcoreMesh(
        core_axis_name="c", subcore_axis_name="s", num_cores=NC),
    scratch_types=[
        pltpu.VMEM((CHUNK,), jnp.int32),
        pltpu.VMEM((VPAD,), jnp.float32),
    ],
    compiler_params=pltpu.CompilerParams(needs_layout_passes=False),
)


def _dense_body(part_ref, lp_ref, w_ref, b_ref, out_ref):
    counts = jnp.sum(part_ref[:, :VOCAB], axis=0, keepdims=True) * (1.0 / L_TOTAL)
    embed = jnp.dot(counts, lp_ref[...], preferred_element_type=jnp.float32)
    out = lax.dot_general(
        embed, w_ref[...], (((1,), (1,)), ((), ())),
        preferred_element_type=jnp.float32)
    out_ref[...] = out + b_ref[...].reshape(1, VOCAB)


_dense = pl.pallas_call(
    _dense_body,
    out_shape=jax.ShapeDtypeStruct((1, VOCAB), jnp.float32),
)


def kernel(x, Lp, W, b):
    x = x.astype(jnp.int32)
    part = _hist(x)
    return _dense(part, Lp, W, b)
